# Initial kernel scaffold; baseline (speedup 1.0000x reference)
import jax
import jax.numpy as jnp
from jax.experimental import pallas as pl


def kernel(x, edge_index, W, attn_l, attn_r):
    raise NotImplementedError("write your pallas kernel here")


# speedup vs baseline: 19.8588x; 19.8588x over previous
import functools

import jax
import jax.numpy as jnp
import numpy as np
from jax import lax
from jax.experimental import pallas as pl
from jax.experimental.pallas import tpu as pltpu
from jax.experimental.pallas import tpu_sc as plsc

N = 10000
D_IN = 128
D_OUT = 32
H = 4
R = 3
E = 200000
NEG = 0.2

EW = 16
NC = 2
NS = 16
NW = NC * NS
K = 64
EPW = 6400
NCH = EPW // K
EP = NW * EPW
RPT = 640
NACC = NS * RPT
ZB = 8
DB = 64

BN = 1000
GRID = N // BN
BN3 = 400
GRID3 = N // BN3


def _tab_body(x_ref, w_ref, q_ref, feat_ref, elr_ref):
    xb = x_ref[...]
    for r in range(R):
        f = jnp.dot(xb, w_ref[r], preferred_element_type=jnp.float32)
        feat_ref[r] = f
        elr_ref[r] = jnp.dot(f, q_ref[r], preferred_element_type=jnp.float32)


def _build_tables(x, W, Q):
    return pl.pallas_call(
        _tab_body,
        grid=(GRID,),
        in_specs=[
            pl.BlockSpec((BN, D_IN), lambda i: (i, 0)),
            pl.BlockSpec((R, D_IN, D_IN), lambda i: (0, 0, 0)),
            pl.BlockSpec((R, D_IN, D_IN), lambda i: (0, 0, 0)),
        ],
        out_specs=[
            pl.BlockSpec((R, BN, D_IN), lambda i: (0, i, 0)),
            pl.BlockSpec((R, BN, D_IN), lambda i: (0, i, 0)),
        ],
        out_shape=[
            jax.ShapeDtypeStruct((R, N, D_IN), jnp.float32),
            jax.ShapeDtypeStruct((R, N, D_IN), jnp.float32),
        ],
    )(x, W, Q)


_SC_MESH = plsc.VectorSubcoreMesh(core_axis_name="c", subcore_axis_name="s")


def _dump_acc(acc, out, c, r, s, bounce):
    base = s * RPT
    nfull = (N - (NS - 1) * RPT) // DB
    rem = (N - (NS - 1) * RPT) - nfull * DB

    def dfull(t, carry):
        pltpu.sync_copy(acc.at[pl.ds(base + t * DB, DB)], bounce)
        pltpu.sync_copy(bounce, out.at[c, r, pl.ds(base + t * DB, DB)])
        return carry

    lax.fori_loop(0, nfull, dfull, 0)
    o1 = base + nfull * DB
    pltpu.sync_copy(acc.at[pl.ds(o1, rem)], bounce.at[pl.ds(0, rem)])
    pltpu.sync_copy(bounce.at[pl.ds(0, rem)], out.at[c, r, pl.ds(o1, rem)])

    @pl.when(s < NS - 1)
    def _rest():
        o2 = base + nfull * DB + rem
        nrest = (RPT - nfull * DB - rem) // DB
        rrem = RPT - nfull * DB - rem - nrest * DB

        def drest(t, carry):
            pltpu.sync_copy(acc.at[pl.ds(o2 + t * DB, DB)], bounce)
            pltpu.sync_copy(bounce, out.at[c, r, pl.ds(o2 + t * DB, DB)])
            return carry

        lax.fori_loop(0, nrest, drest, 0)
        o3 = o2 + nrest * DB
        pltpu.sync_copy(acc.at[pl.ds(o3, rrem)], bounce.at[pl.ds(0, rrem)])
        pltpu.sync_copy(bounce.at[pl.ds(0, rrem)],
                        out.at[c, r, pl.ds(o3, rrem)])


@functools.partial(
    pl.kernel,
    out_type=(
        jax.ShapeDtypeStruct((NC, R, N, D_IN), jnp.float32),
        jax.ShapeDtypeStruct((R, NW, NCH, K, EW), jnp.float32),
    ),
    mesh=_SC_MESH,
    compiler_params=pltpu.CompilerParams(needs_layout_passes=False),
    scratch_types=[
        pltpu.VMEM((K,), jnp.int32),
        pltpu.VMEM((1, K), jnp.int32),
        pltpu.VMEM((K,), jnp.int32),
        pltpu.VMEM((K, D_IN), jnp.float32),
        pltpu.VMEM((K, D_IN), jnp.float32),
        pltpu.VMEM((K, D_IN), jnp.float32),
        pltpu.VMEM((K, EW), jnp.float32),
        pltpu.VMEM((ZB, D_IN), jnp.float32),
        pltpu.VMEM_SHARED((NACC, D_IN), jnp.float32),
        pltpu.SemaphoreType.DMA,
        pltpu.SemaphoreType.DMA,
        pltpu.SemaphoreType.DMA,
    ],
)
def _num_kernel(srcoff, dstoff, dstacc, feattab, elrtab,
                pnum, pvals,
                srci, dsta, dio, A, EL, ER, P, Z,
                accn, sem1, sem2, sem3):
    c = lax.axis_index("c")
    s = lax.axis_index("s")
    w = c * NS + s

    zv = jnp.zeros((16,), jnp.float32)

    def zrow(i, carry):
        for j in range(D_IN // 16):
            Z[i, j * 16:(j + 1) * 16] = zv
        return carry

    lax.fori_loop(0, ZB, zrow, 0)

    lane = lax.iota(jnp.int32, 16)
    shift4 = (lane + 4) % 16
    s4 = jnp.reshape(shift4, (16, 1))
    hsplat = [jnp.full((16, 1), h, jnp.int32) for h in range(H)]
    gdn = lax.GatherDimensionNumbers(
        offset_dims=(), collapsed_slice_dims=(0,), start_index_map=(0,))

    def lane_bcast(vec, idx):
        return lax.gather(vec, idx, gdn, (1,),
                          mode=lax.GatherScatterMode.PROMISE_IN_BOUNDS)

    for r in range(R):
        def zcp(t, carry):
            pltpu.sync_copy(Z, accn.at[pl.ds(s * RPT + t * ZB, ZB)])
            return carry

        lax.fori_loop(0, RPT // ZB, zcp, 0)
        plsc.subcore_barrier()

        def chunk(ci, carry):
            pltpu.sync_copy(srcoff.at[r, w, ci], srci)
            pltpu.sync_copy(dstoff.at[r, w, ci], dio)
            pltpu.sync_copy(dstacc.at[r, w, pl.ds(ci, 1)], dsta)
            ga = pltpu.async_copy(feattab.at[srci], A, sem1)
            gb = pltpu.async_copy(elrtab.at[srci], EL, sem2)
            gc = pltpu.async_copy(elrtab.at[dio], ER, sem3)
            ga.wait()
            gb.wait()
            gc.wait()

            def edge(k, inner):
                el = EL[k, 0:16]
                er = lane_bcast(ER[k, 0:16], s4)
                v = el + er
                e = jnp.where(v >= 0, v, NEG * v)
                p = jnp.exp(e)
                P[k, 0:16] = p
                for q in range(D_IN // 16):
                    m = lane_bcast(p, hsplat[q // 2])
                    A[k, q * 16:(q + 1) * 16] = (
                        A[k, q * 16:(q + 1) * 16] * m)
                return inner

            lax.fori_loop(0, K, edge, 0)
            pltpu.sync_copy(A, accn.at[dsta.at[0]], add=True)
            pltpu.sync_copy(P, pvals.at[r, w, ci])
            return carry

        lax.fori_loop(0, NCH, chunk, 0)
        plsc.subcore_barrier()
        _dump_acc(accn, pnum, c, r, s, A)
        plsc.subcore_barrier()


@functools.partial(
    pl.kernel,
    out_type=jax.ShapeDtypeStruct((NC, R, N, D_IN), jnp.float32),
    mesh=_SC_MESH,
    compiler_params=pltpu.CompilerParams(needs_layout_passes=False),
    scratch_types=[
        pltpu.VMEM((1, K), jnp.int32),
        pltpu.VMEM((K, EW), jnp.float32),
        pltpu.VMEM((K, D_IN), jnp.float32),
        pltpu.VMEM((DB, D_IN), jnp.float32),
        pltpu.VMEM((ZB, D_IN), jnp.float32),
        pltpu.VMEM_SHARED((NACC, D_IN), jnp.float32),
    ],
)
def _den_kernel(dstacc, pvals,
                pden,
                dsta, P16, P128, DPB, Z,
                accd):
    c = lax.axis_index("c")
    s = lax.axis_index("s")
    w = c * NS + s

    zv = jnp.zeros((16,), jnp.float32)

    def zrow(i, carry):
        for j in range(D_IN // 16):
            Z[i, j * 16:(j + 1) * 16] = zv
        return carry

    lax.fori_loop(0, ZB, zrow, 0)

    def zp(k, carry):
        for j in range(D_IN // 16):
            P128[k, j * 16:(j + 1) * 16] = zv
        return carry

    lax.fori_loop(0, K, zp, 0)

    for r in range(R):
        def zcp(t, carry):
            pltpu.sync_copy(Z, accd.at[pl.ds(s * RPT + t * ZB, ZB)])
            return carry

        lax.fori_loop(0, RPT // ZB, zcp, 0)
        plsc.subcore_barrier()

        def chunk(ci, carry):
            pltpu.sync_copy(dstacc.at[r, w, pl.ds(ci, 1)], dsta)
            pltpu.sync_copy(pvals.at[r, w, ci], P16)

            def row(k, inner):
                P128[k, 0:16] = P16[k, 0:16]
                return inner

            lax.fori_loop(0, K, row, 0)
            pltpu.sync_copy(P128, accd.at[dsta.at[0]], add=True)
            return carry

        lax.fori_loop(0, NCH, chunk, 0)
        plsc.subcore_barrier()
        _dump_acc(accd, pden, c, r, s, DPB)
        plsc.subcore_barrier()


def _att_body(pnum_ref, pden_ref, x_ref, out_ref, am_ref):
    i = pl.program_id(0)
    num = pnum_ref[0] + pnum_ref[1]
    den = pden_ref[0] + pden_ref[1]
    fcs = []
    for r in range(R):
        for h in range(H):
            nrh = num[r, :, h * D_OUT:(h + 1) * D_OUT]
            drh = den[r, :, h:h + 1]
            fcs.append(jnp.where(drh > 0, nrh / drh, 0.0))
    fc = jnp.stack(fcs, axis=1)
    inv = 1.0 / np.sqrt(D_OUT)
    rows = []
    for l in range(R * H):
        rows.append(jnp.sum(fc * fc[:, l:l + 1, :], axis=-1) * inv)
    scores = jnp.stack(rows, axis=1)
    mx = jnp.max(scores, axis=-1, keepdims=True)
    ex = jnp.exp(scores - mx)
    attn = ex / jnp.sum(ex, axis=-1, keepdims=True)
    cm = jnp.mean(attn, axis=1)
    rst = jnp.zeros((BN3, D_OUT), jnp.float32)
    for m in range(R * H):
        rst = rst + cm[:, m:m + 1] * fc[:, m, :]
    out_ref[...] = jnp.concatenate([x_ref[...], rst], axis=1)
    part_am = jnp.sum(attn, axis=0) * (1.0 / N)

    @pl.when(i == 0)
    def _init():
        am_ref[...] = part_am

    @pl.when(i > 0)
    def _acc():
        am_ref[...] = am_ref[...] + part_am


def _attention(pnum, pden, x):
    L = R * H
    return pl.pallas_call(
        _att_body,
        grid=(GRID3,),
        in_specs=[
            pl.BlockSpec((NC, R, BN3, D_IN), lambda i: (0, 0, i, 0)),
            pl.BlockSpec((NC, R, BN3, D_IN), lambda i: (0, 0, i, 0)),
            pl.BlockSpec((BN3, D_IN), lambda i: (i, 0)),
        ],
        out_specs=[
            pl.BlockSpec((BN3, D_IN + D_OUT), lambda i: (i, 0)),
            pl.BlockSpec((L, L), lambda i: (0, 0)),
        ],
        out_shape=[
            jax.ShapeDtypeStruct((N, D_IN + D_OUT), jnp.float32),
            jax.ShapeDtypeStruct((L, L), jnp.float32),
        ],
    )(pnum, pden, x)


def kernel(x, edge_index, W, attn_l, attn_r):
    mask = (jnp.arange(D_IN)[:, None] // D_OUT == jnp.arange(H)[None, :])
    mask = mask.astype(jnp.float32)
    Pl = attn_l.reshape(R, D_IN)[:, :, None] * mask[None]
    Pr = attn_r.reshape(R, D_IN)[:, :, None] * mask[None]
    zpad = jnp.zeros((R, D_IN, D_IN - 2 * H), jnp.float32)
    Q = jnp.concatenate([Pl, Pr, zpad], axis=2)

    ei = edge_index.reshape(2, R, E)
    roff = (jnp.arange(R, dtype=jnp.int32) * N)[:, None]
    pad = EP - E
    src = jnp.concatenate(
        [ei[0] + roff, jnp.zeros((R, pad), jnp.int32) + roff], axis=1)
    dstg = jnp.minimum(
        jnp.concatenate(
            [ei[1] + roff, jnp.full((R, pad), N, jnp.int32) + roff], axis=1),
        R * N - 1)
    dsts = jnp.concatenate(
        [ei[1], jnp.full((R, pad), N, jnp.int32)], axis=1)
    srcoff = src.reshape(R, NW, NCH, K)
    dstoff = dstg.reshape(R, NW, NCH, K)
    dstacc = dsts.reshape(R, NW, NCH, K)

    feattab, elrtab = _build_tables(x, W, Q)
    feattab = feattab.reshape(R * N, D_IN)
    elrtab = elrtab.reshape(R * N, D_IN)

    pnum, pvals = _num_kernel(srcoff, dstoff, dstacc, feattab, elrtab)
    pden = _den_kernel(dstacc, pvals)

    out, attn_map = _attention(pnum, pden, x)
    return out, attn_map


# speedup vs baseline: 22.2951x; 1.1227x over previous
import functools

import jax
import jax.numpy as jnp
import numpy as np
from jax import lax
from jax.experimental import pallas as pl
from jax.experimental.pallas import tpu as pltpu
from jax.experimental.pallas import tpu_sc as plsc

N = 10000
D_IN = 128
D_OUT = 32
H = 4
R = 3
E = 200000
NEG = 0.2

EW = 16
NC = 2
NS = 16
NW = NC * NS
K = 64
EPW = 6400
NCH = EPW // K
EP = NW * EPW
RPT = 640
NACC = NS * RPT
ZB = 8
DB = 64

BN = 1000
GRID = N // BN
BN3 = 400
GRID3 = N // BN3


def _tab_body(x_ref, w_ref, q_ref, feat_ref, elr_ref):
    xb = x_ref[...]
    for r in range(R):
        f = jnp.dot(xb, w_ref[r], preferred_element_type=jnp.float32)
        feat_ref[r] = f
        elr_ref[r] = jnp.dot(f, q_ref[r], preferred_element_type=jnp.float32)


def _build_tables(x, W, Q):
    return pl.pallas_call(
        _tab_body,
        grid=(GRID,),
        in_specs=[
            pl.BlockSpec((BN, D_IN), lambda i: (i, 0)),
            pl.BlockSpec((R, D_IN, D_IN), lambda i: (0, 0, 0)),
            pl.BlockSpec((R, D_IN, D_IN), lambda i: (0, 0, 0)),
        ],
        out_specs=[
            pl.BlockSpec((R, BN, D_IN), lambda i: (0, i, 0)),
            pl.BlockSpec((R, BN, D_IN), lambda i: (0, i, 0)),
        ],
        out_shape=[
            jax.ShapeDtypeStruct((R, N, D_IN), jnp.float32),
            jax.ShapeDtypeStruct((R, N, D_IN), jnp.float32),
        ],
    )(x, W, Q)


_SC_MESH = plsc.VectorSubcoreMesh(core_axis_name="c", subcore_axis_name="s")


def _dump_acc(acc, out, c, r, s, bounce):
    base = s * RPT
    nfull = (N - (NS - 1) * RPT) // DB
    rem = (N - (NS - 1) * RPT) - nfull * DB

    def dfull(t, carry):
        pltpu.sync_copy(acc.at[pl.ds(base + t * DB, DB)], bounce)
        pltpu.sync_copy(bounce, out.at[c, r, pl.ds(base + t * DB, DB)])
        return carry

    lax.fori_loop(0, nfull, dfull, 0)
    o1 = base + nfull * DB
    pltpu.sync_copy(acc.at[pl.ds(o1, rem)], bounce.at[pl.ds(0, rem)])
    pltpu.sync_copy(bounce.at[pl.ds(0, rem)], out.at[c, r, pl.ds(o1, rem)])

    @pl.when(s < NS - 1)
    def _rest():
        o2 = base + nfull * DB + rem
        nrest = (RPT - nfull * DB - rem) // DB
        rrem = RPT - nfull * DB - rem - nrest * DB

        def drest(t, carry):
            pltpu.sync_copy(acc.at[pl.ds(o2 + t * DB, DB)], bounce)
            pltpu.sync_copy(bounce, out.at[c, r, pl.ds(o2 + t * DB, DB)])
            return carry

        lax.fori_loop(0, nrest, drest, 0)
        o3 = o2 + nrest * DB
        pltpu.sync_copy(acc.at[pl.ds(o3, rrem)], bounce.at[pl.ds(0, rrem)])
        pltpu.sync_copy(bounce.at[pl.ds(0, rrem)],
                        out.at[c, r, pl.ds(o3, rrem)])


@functools.partial(
    pl.kernel,
    out_type=(
        jax.ShapeDtypeStruct((NC, R, N, D_IN), jnp.float32),
        jax.ShapeDtypeStruct((R, NW, NCH, K, EW), jnp.float32),
    ),
    mesh=_SC_MESH,
    compiler_params=pltpu.CompilerParams(needs_layout_passes=False),
    scratch_types=[
        pltpu.VMEM((K,), jnp.int32),
        pltpu.VMEM((1, K), jnp.int32),
        pltpu.VMEM((K,), jnp.int32),
        pltpu.VMEM((K, D_IN), jnp.float32),
        pltpu.VMEM((K, D_IN), jnp.float32),
        pltpu.VMEM((K, D_IN), jnp.float32),
        pltpu.VMEM((K, EW), jnp.float32),
        pltpu.VMEM((ZB, D_IN), jnp.float32),
        pltpu.VMEM_SHARED((NACC, D_IN), jnp.float32),
        pltpu.SemaphoreType.DMA,
        pltpu.SemaphoreType.DMA,
        pltpu.SemaphoreType.DMA,
    ],
)
def _num_kernel(srcoff, dstoff, dstacc, feattab, elrtab,
                pnum, pvals,
                srci, dsta, dio, A, EL, ER, P, Z,
                accn, sem1, sem2, sem3):
    c = lax.axis_index("c")
    s = lax.axis_index("s")
    w = c * NS + s

    zv = jnp.zeros((16,), jnp.float32)

    def zrow(i, carry):
        for j in range(D_IN // 16):
            Z[i, j * 16:(j + 1) * 16] = zv
        return carry

    lax.fori_loop(0, ZB, zrow, 0)

    lane = lax.iota(jnp.int32, 16)
    shift4 = (lane + 4) % 16
    s4 = jnp.reshape(shift4, (16, 1))
    hsplat = [jnp.full((16, 1), h, jnp.int32) for h in range(H)]
    gdn = lax.GatherDimensionNumbers(
        offset_dims=(), collapsed_slice_dims=(0,), start_index_map=(0,))

    def lane_bcast(vec, idx):
        return lax.gather(vec, idx, gdn, (1,),
                          mode=lax.GatherScatterMode.PROMISE_IN_BOUNDS)

    for r in range(R):
        def zcp(t, carry):
            pltpu.sync_copy(Z, accn.at[pl.ds(s * RPT + t * ZB, ZB)])
            return carry

        lax.fori_loop(0, RPT // ZB, zcp, 0)
        plsc.subcore_barrier()

        def chunk(ci, carry):
            ia = pltpu.async_copy(srcoff.at[r, w, ci], srci, sem1)
            ib = pltpu.async_copy(dstoff.at[r, w, ci], dio, sem2)
            ic = pltpu.async_copy(dstacc.at[r, w, pl.ds(ci, 1)], dsta, sem3)
            ia.wait()
            ib.wait()
            ic.wait()
            ga = pltpu.async_copy(feattab.at[srci], A, sem1)
            gb = pltpu.async_copy(elrtab.at[srci], EL, sem2)
            gc = pltpu.async_copy(elrtab.at[dio], ER, sem3)
            ga.wait()
            gb.wait()
            gc.wait()

            def edge(k, inner):
                el = EL[k, 0:16]
                er = lane_bcast(ER[k, 0:16], s4)
                v = el + er
                e = jnp.where(v >= 0, v, NEG * v)
                p = jnp.exp(e)
                P[k, 0:16] = p
                for q in range(D_IN // 16):
                    m = lane_bcast(p, hsplat[q // 2])
                    A[k, q * 16:(q + 1) * 16] = (
                        A[k, q * 16:(q + 1) * 16] * m)
                return inner

            lax.fori_loop(0, K, edge, 0)
            sa = pltpu.async_copy(A, accn.at[dsta.at[0]], sem1, add=True)
            sb = pltpu.async_copy(P, pvals.at[r, w, ci], sem2)
            sa.wait()
            sb.wait()
            return carry

        lax.fori_loop(0, NCH, chunk, 0)
        plsc.subcore_barrier()
        _dump_acc(accn, pnum, c, r, s, A)
        plsc.subcore_barrier()


@functools.partial(
    pl.kernel,
    out_type=jax.ShapeDtypeStruct((NC, R, N, D_IN), jnp.float32),
    mesh=_SC_MESH,
    compiler_params=pltpu.CompilerParams(needs_layout_passes=False),
    scratch_types=[
        pltpu.VMEM((1, K), jnp.int32),
        pltpu.VMEM((K, EW), jnp.float32),
        pltpu.VMEM((K, D_IN), jnp.float32),
        pltpu.VMEM((DB, D_IN), jnp.float32),
        pltpu.VMEM((ZB, D_IN), jnp.float32),
        pltpu.VMEM_SHARED((NACC, D_IN), jnp.float32),
        pltpu.SemaphoreType.DMA,
        pltpu.SemaphoreType.DMA,
    ],
)
def _den_kernel(dstacc, pvals,
                pden,
                dsta, P16, P128, DPB, Z,
                accd, semd1, semd2):
    c = lax.axis_index("c")
    s = lax.axis_index("s")
    w = c * NS + s

    zv = jnp.zeros((16,), jnp.float32)

    def zrow(i, carry):
        for j in range(D_IN // 16):
            Z[i, j * 16:(j + 1) * 16] = zv
        return carry

    lax.fori_loop(0, ZB, zrow, 0)

    def zp(k, carry):
        for j in range(D_IN // 16):
            P128[k, j * 16:(j + 1) * 16] = zv
        return carry

    lax.fori_loop(0, K, zp, 0)

    for r in range(R):
        def zcp(t, carry):
            pltpu.sync_copy(Z, accd.at[pl.ds(s * RPT + t * ZB, ZB)])
            return carry

        lax.fori_loop(0, RPT // ZB, zcp, 0)
        plsc.subcore_barrier()

        def chunk(ci, carry):
            ia = pltpu.async_copy(dstacc.at[r, w, pl.ds(ci, 1)], dsta, semd1)
            ib = pltpu.async_copy(pvals.at[r, w, ci], P16, semd2)
            ia.wait()
            ib.wait()

            def row(k, inner):
                P128[k, 0:16] = P16[k, 0:16]
                return inner

            lax.fori_loop(0, K, row, 0)
            pltpu.sync_copy(P128, accd.at[dsta.at[0]], add=True)
            return carry

        lax.fori_loop(0, NCH, chunk, 0)
        plsc.subcore_barrier()
        _dump_acc(accd, pden, c, r, s, DPB)
        plsc.subcore_barrier()


def _att_body(pnum_ref, pden_ref, x_ref, out_ref, am_ref):
    i = pl.program_id(0)
    num = pnum_ref[0] + pnum_ref[1]
    den = pden_ref[0] + pden_ref[1]
    fcs = []
    for r in range(R):
        for h in range(H):
            nrh = num[r, :, h * D_OUT:(h + 1) * D_OUT]
            drh = den[r, :, h:h + 1]
            fcs.append(jnp.where(drh > 0, nrh / drh, 0.0))
    fc = jnp.stack(fcs, axis=1)
    inv = 1.0 / np.sqrt(D_OUT)
    rows = []
    for l in range(R * H):
        rows.append(jnp.sum(fc * fc[:, l:l + 1, :], axis=-1) * inv)
    scores = jnp.stack(rows, axis=1)
    mx = jnp.max(scores, axis=-1, keepdims=True)
    ex = jnp.exp(scores - mx)
    attn = ex / jnp.sum(ex, axis=-1, keepdims=True)
    cm = jnp.mean(attn, axis=1)
    rst = jnp.zeros((BN3, D_OUT), jnp.float32)
    for m in range(R * H):
        rst = rst + cm[:, m:m + 1] * fc[:, m, :]
    out_ref[...] = jnp.concatenate([x_ref[...], rst], axis=1)
    part_am = jnp.sum(attn, axis=0) * (1.0 / N)

    @pl.when(i == 0)
    def _init():
        am_ref[...] = part_am

    @pl.when(i > 0)
    def _acc():
        am_ref[...] = am_ref[...] + part_am


def _attention(pnum, pden, x):
    L = R * H
    return pl.pallas_call(
        _att_body,
        grid=(GRID3,),
        in_specs=[
            pl.BlockSpec((NC, R, BN3, D_IN), lambda i: (0, 0, i, 0)),
            pl.BlockSpec((NC, R, BN3, D_IN), lambda i: (0, 0, i, 0)),
            pl.BlockSpec((BN3, D_IN), lambda i: (i, 0)),
        ],
        out_specs=[
            pl.BlockSpec((BN3, D_IN + D_OUT), lambda i: (i, 0)),
            pl.BlockSpec((L, L), lambda i: (0, 0)),
        ],
        out_shape=[
            jax.ShapeDtypeStruct((N, D_IN + D_OUT), jnp.float32),
            jax.ShapeDtypeStruct((L, L), jnp.float32),
        ],
    )(pnum, pden, x)


def kernel(x, edge_index, W, attn_l, attn_r):
    mask = (jnp.arange(D_IN)[:, None] // D_OUT == jnp.arange(H)[None, :])
    mask = mask.astype(jnp.float32)
    Pl = attn_l.reshape(R, D_IN)[:, :, None] * mask[None]
    Pr = attn_r.reshape(R, D_IN)[:, :, None] * mask[None]
    zpad = jnp.zeros((R, D_IN, D_IN - 2 * H), jnp.float32)
    Q = jnp.concatenate([Pl, Pr, zpad], axis=2)

    ei = edge_index.reshape(2, R, E)
    roff = (jnp.arange(R, dtype=jnp.int32) * N)[:, None]
    pad = EP - E
    src = jnp.concatenate(
        [ei[0] + roff, jnp.zeros((R, pad), jnp.int32) + roff], axis=1)
    dstg = jnp.minimum(
        jnp.concatenate(
            [ei[1] + roff, jnp.full((R, pad), N, jnp.int32) + roff], axis=1),
        R * N - 1)
    dsts = jnp.concatenate(
        [ei[1], jnp.full((R, pad), N, jnp.int32)], axis=1)
    srcoff = src.reshape(R, NW, NCH, K)
    dstoff = dstg.reshape(R, NW, NCH, K)
    dstacc = dsts.reshape(R, NW, NCH, K)

    feattab, elrtab = _build_tables(x, W, Q)
    feattab = feattab.reshape(R * N, D_IN)
    elrtab = elrtab.reshape(R * N, D_IN)

    pnum, pvals = _num_kernel(srcoff, dstoff, dstacc, feattab, elrtab)
    pden = _den_kernel(dstacc, pvals)

    out, attn_map = _attention(pnum, pden, x)
    return out, attn_map
